# baseline (speedup 1.0000x reference)
import functools

import jax
import jax.numpy as jnp
from jax import lax
from jax.experimental import pallas as pl
from jax.experimental.pallas import tpu as pltpu
from jax.experimental.pallas import tpu_sc as plsc

N = 10000
E = 320000
F = 128
H = 8
D = 16
HD = H * D
ROW = HD + 2 * H
NEG_SLOPE = 0.2

NC = 2
NS = 16
NW = NC * NS
CHUNK = 80
EPT = E // NW
NCHUNKS = EPT // CHUNK
RPT = N // NS


def _prep_body(x_ref, w_ref, al_ref, ar_ref, hs_ref, er_ref):
    h = jnp.dot(x_ref[...], w_ref[...], preferred_element_type=jnp.float32)
    hs_ref[:, :HD] = h
    row = lax.broadcasted_iota(jnp.int32, (2 * H, F), 0)
    col = lax.broadcasted_iota(jnp.int32, (2 * H, F), 1)
    sel = (col // D) == row
    alt = jnp.where(sel, al_ref[...], 0.0)
    art = jnp.where(sel, ar_ref[...], 0.0)
    dn = (((1,), (1,)), ((), ()))
    pad = jnp.where(lax.broadcasted_iota(jnp.int32, (1, 2 * H), 1) < H, 0.0, -1e30)
    hs_ref[:, HD:ROW] = lax.dot_general(h, alt, dn, preferred_element_type=jnp.float32) + pad
    er_ref[...] = lax.dot_general(h, art, dn, preferred_element_type=jnp.float32) + pad


def _fin_body(p0_ref, p1_ref, o_ref):
    a = p0_ref[...] + p1_ref[...]
    msg = a[:, :HD]
    den = a[:, HD:HD + H]
    recip = 1.0 / (den + 1e-9)
    r8 = lax.broadcasted_iota(jnp.int32, (H, HD), 0)
    c8 = lax.broadcasted_iota(jnp.int32, (H, HD), 1)
    expand = ((c8 // D) == r8).astype(jnp.float32)
    o_ref[...] = msg * jnp.dot(recip, expand, preferred_element_type=jnp.float32)


def _edge_compute(hs_v, er_v, out_v):
    @plsc.parallel_loop(0, CHUNK, step=1, unroll=16)
    def _body(e):
        el = hs_v[e, pl.ds(HD, 16)]
        er = er_v[e, pl.ds(0, 16)]
        s = el + er
        ex = jnp.exp(jnp.maximum(s, NEG_SLOPE * s))
        out_v[e, pl.ds(HD, 16)] = ex
        for hh in range(H):
            sx = jnp.broadcast_to(ex[hh], (16,))
            hv = hs_v[e, pl.ds(hh * D, 16)]
            out_v[e, pl.ds(hh * D, 16)] = hv * sx


def _edge_body(src_hbm, dst_hbm, hs_hbm, er_hbm, out_hbm,
               src_a, dst_a, hs_a, er_a,
               src_b, dst_b, hs_b, er_b,
               out_v, acc,
               hsem_a, esem_a, hsem_b, esem_b, isem_a, isem_b):
    cid = lax.axis_index("c")
    sid = lax.axis_index("s")
    wid = cid * NS + sid
    base = wid * EPT

    @plsc.parallel_loop(0, CHUNK, step=1, unroll=8)
    def _zrow(r):
        for c2 in range(ROW // 16):
            out_v[r, pl.ds(c2 * 16, 16)] = jnp.zeros((16,), jnp.float32)

    for k in range(RPT // CHUNK):
        pltpu.sync_copy(out_v, acc.at[pl.ds(sid * RPT + k * CHUNK, CHUNK)])
    _rem = RPT - (RPT // CHUNK) * CHUNK
    if _rem:
        pltpu.sync_copy(out_v.at[pl.ds(0, _rem)],
                        acc.at[pl.ds(sid * RPT + (RPT // CHUNK) * CHUNK, _rem)])
    plsc.subcore_barrier()

    def fetch_idx(c, sv, dv, isem):
        off = base + jnp.minimum(c, NCHUNKS - 1) * CHUNK
        pltpu.async_copy(src_hbm.at[pl.ds(off, CHUNK)], sv, isem)
        pltpu.async_copy(dst_hbm.at[pl.ds(off, CHUNK)], dv, isem)

    def wait_idx(sv, dv, isem):
        pltpu.make_async_copy(src_hbm.at[pl.ds(0, CHUNK)], sv, isem).wait()
        pltpu.make_async_copy(dst_hbm.at[pl.ds(0, CHUNK)], dv, isem).wait()

    def start_gathers(sv, dv, hv, ev, hsem, esem):
        pltpu.async_copy(hs_hbm.at[sv], hv, hsem)
        pltpu.async_copy(er_hbm.at[dv], ev, esem)

    def wait_gathers(sv, dv, hv, ev, hsem, esem):
        pltpu.make_async_copy(hs_hbm.at[sv], hv, hsem).wait()
        pltpu.make_async_copy(er_hbm.at[dv], ev, esem).wait()

    fetch_idx(0, src_a, dst_a, isem_a)
    wait_idx(src_a, dst_a, isem_a)
    start_gathers(src_a, dst_a, hs_a, er_a, hsem_a, esem_a)
    fetch_idx(1, src_b, dst_b, isem_b)

    def pair_body(j, carry):
        wait_idx(src_b, dst_b, isem_b)
        start_gathers(src_b, dst_b, hs_b, er_b, hsem_b, esem_b)
        wait_gathers(src_a, dst_a, hs_a, er_a, hsem_a, esem_a)
        _edge_compute(hs_a, er_a, out_v)
        pltpu.sync_copy(out_v, acc.at[dst_a], add=True)
        fetch_idx(2 * j + 2, src_a, dst_a, isem_a)

        wait_idx(src_a, dst_a, isem_a)
        start_gathers(src_a, dst_a, hs_a, er_a, hsem_a, esem_a)
        wait_gathers(src_b, dst_b, hs_b, er_b, hsem_b, esem_b)
        _edge_compute(hs_b, er_b, out_v)
        pltpu.sync_copy(out_v, acc.at[dst_b], add=True)
        fetch_idx(2 * j + 3, src_b, dst_b, isem_b)
        return carry

    lax.fori_loop(0, NCHUNKS // 2, pair_body, 0)

    wait_idx(src_b, dst_b, isem_b)
    wait_gathers(src_a, dst_a, hs_a, er_a, hsem_a, esem_a)
    _edge_compute(hs_a, er_a, out_v)
    pltpu.sync_copy(out_v, acc.at[dst_a], add=True)

    plsc.subcore_barrier()
    pltpu.sync_copy(acc.at[pl.ds(sid * RPT, RPT)],
                    out_hbm.at[pl.ds(cid * N + sid * RPT, RPT)])


@jax.jit
def kernel(x, edge_index, W, attn_l, attn_r):
    srcp = edge_index[0].astype(jnp.int32)
    dstp = edge_index[1].astype(jnp.int32)

    bn = 1000
    grid = N // bn
    hs, er = pl.pallas_call(
        _prep_body,
        grid=(grid,),
        in_specs=[
            pl.BlockSpec((bn, F), lambda i: (i, 0)),
            pl.BlockSpec((F, HD), lambda i: (0, 0)),
            pl.BlockSpec((1, F), lambda i: (0, 0)),
            pl.BlockSpec((1, F), lambda i: (0, 0)),
        ],
        out_specs=[
            pl.BlockSpec((bn, ROW), lambda i: (i, 0)),
            pl.BlockSpec((bn, 2 * H), lambda i: (i, 0)),
        ],
        out_shape=[
            jax.ShapeDtypeStruct((N, ROW), jnp.float32),
            jax.ShapeDtypeStruct((N, 2 * H), jnp.float32),
        ],
    )(x, W, attn_l.reshape(1, F), attn_r.reshape(1, F))

    mesh = plsc.VectorSubcoreMesh(core_axis_name="c", subcore_axis_name="s",
                                  num_cores=NC, num_subcores=NS)
    partials = pl.kernel(
        _edge_body,
        out_type=jax.ShapeDtypeStruct((NC * N, ROW), jnp.float32),
        mesh=mesh,
        scratch_types=(
            [pltpu.VMEM((CHUNK,), jnp.int32),
             pltpu.VMEM((CHUNK,), jnp.int32),
             pltpu.VMEM((CHUNK, ROW), jnp.float32),
             pltpu.VMEM((CHUNK, 2 * H), jnp.float32)] * 2 +
            [pltpu.VMEM((CHUNK, ROW), jnp.float32),
             pltpu.VMEM_SHARED((N, ROW), jnp.float32),
             pltpu.SemaphoreType.DMA,
             pltpu.SemaphoreType.DMA,
             pltpu.SemaphoreType.DMA,
             pltpu.SemaphoreType.DMA,
             pltpu.SemaphoreType.DMA,
             pltpu.SemaphoreType.DMA]
        ),
        compiler_params=pltpu.CompilerParams(use_tc_tiling_on_sc=False),
    )(srcp, dstp, hs, er)

    feat = pl.pallas_call(
        _fin_body,
        grid=(grid,),
        in_specs=[
            pl.BlockSpec((bn, ROW), lambda i: (i, 0)),
            pl.BlockSpec((bn, ROW), lambda i: (i + grid, 0)),
        ],
        out_specs=pl.BlockSpec((bn, HD), lambda i: (i, 0)),
        out_shape=jax.ShapeDtypeStruct((N, HD), jnp.float32),
    )(partials, partials)
    return feat


# speedup vs baseline: 1.0163x; 1.0163x over previous
import functools

import jax
import jax.numpy as jnp
from jax import lax
from jax.experimental import pallas as pl
from jax.experimental.pallas import tpu as pltpu
from jax.experimental.pallas import tpu_sc as plsc

N = 10000
E = 320000
F = 128
H = 8
D = 16
HD = H * D
ROW = HD + 2 * H
NEG_SLOPE = 0.2

NC = 2
NS = 16
NW = NC * NS
CHUNK = 80
EPT = E // NW
NCHUNKS = EPT // CHUNK
RPT = N // NS


def _prep_body(x_ref, w_ref, al_ref, ar_ref, hs_ref, er_ref):
    h = jnp.dot(x_ref[...], w_ref[...], preferred_element_type=jnp.float32)
    hs_ref[:, :HD] = h
    row = lax.broadcasted_iota(jnp.int32, (2 * H, F), 0)
    col = lax.broadcasted_iota(jnp.int32, (2 * H, F), 1)
    sel = (col // D) == row
    alt = jnp.where(sel, al_ref[...], 0.0)
    art = jnp.where(sel, ar_ref[...], 0.0)
    dn = (((1,), (1,)), ((), ()))
    pad = jnp.where(lax.broadcasted_iota(jnp.int32, (1, 2 * H), 1) < H, 0.0, -1e30)
    hs_ref[:, HD:ROW] = lax.dot_general(h, alt, dn, preferred_element_type=jnp.float32) + pad
    er_ref[...] = lax.dot_general(h, art, dn, preferred_element_type=jnp.float32) + pad


def _fin_body(p0_ref, p1_ref, o_ref):
    a = p0_ref[...] + p1_ref[...]
    msg = a[:, :HD]
    den = a[:, HD:HD + H]
    recip = 1.0 / (den + 1e-9)
    r8 = lax.broadcasted_iota(jnp.int32, (H, HD), 0)
    c8 = lax.broadcasted_iota(jnp.int32, (H, HD), 1)
    expand = ((c8 // D) == r8).astype(jnp.float32)
    o_ref[...] = msg * jnp.dot(recip, expand, preferred_element_type=jnp.float32)


def _edge_compute(hs_v, er_v, out_v):
    @plsc.parallel_loop(0, CHUNK, step=1, unroll=8)
    def _body(e):
        el = hs_v[e, pl.ds(HD, 16)]
        er = er_v[e, pl.ds(0, 16)]
        s = el + er
        ex = jnp.exp(jnp.maximum(s, NEG_SLOPE * s))
        out_v[e, pl.ds(HD, 16)] = ex
        for hh in range(H):
            sx = jnp.broadcast_to(ex[hh], (16,))
            hv = hs_v[e, pl.ds(hh * D, 16)]
            out_v[e, pl.ds(hh * D, 16)] = hv * sx


def _edge_body(src_hbm, dst_hbm, hs_hbm, er_hbm, out_hbm,
               src_a, dst_a, hs_a, er_a,
               src_b, dst_b, hs_b, er_b,
               out_v, acc,
               hsem_a, esem_a, hsem_b, esem_b, isem_a, isem_b):
    cid = lax.axis_index("c")
    sid = lax.axis_index("s")
    wid = cid * NS + sid
    base = wid * EPT

    @plsc.parallel_loop(0, CHUNK, step=1, unroll=8)
    def _zrow(r):
        for c2 in range(ROW // 16):
            out_v[r, pl.ds(c2 * 16, 16)] = jnp.zeros((16,), jnp.float32)

    for k in range(RPT // CHUNK):
        pltpu.sync_copy(out_v, acc.at[pl.ds(sid * RPT + k * CHUNK, CHUNK)])
    _rem = RPT - (RPT // CHUNK) * CHUNK
    if _rem:
        pltpu.sync_copy(out_v.at[pl.ds(0, _rem)],
                        acc.at[pl.ds(sid * RPT + (RPT // CHUNK) * CHUNK, _rem)])
    plsc.subcore_barrier()

    def fetch_idx(c, sv, dv, isem):
        off = base + jnp.minimum(c, NCHUNKS - 1) * CHUNK
        pltpu.async_copy(src_hbm.at[pl.ds(off, CHUNK)], sv, isem)
        pltpu.async_copy(dst_hbm.at[pl.ds(off, CHUNK)], dv, isem)

    def wait_idx(sv, dv, isem):
        pltpu.make_async_copy(src_hbm.at[pl.ds(0, CHUNK)], sv, isem).wait()
        pltpu.make_async_copy(dst_hbm.at[pl.ds(0, CHUNK)], dv, isem).wait()

    def start_gathers(sv, dv, hv, ev, hsem, esem):
        pltpu.async_copy(hs_hbm.at[sv], hv, hsem)
        pltpu.async_copy(er_hbm.at[dv], ev, esem)

    def wait_gathers(sv, dv, hv, ev, hsem, esem):
        pltpu.make_async_copy(hs_hbm.at[sv], hv, hsem).wait()
        pltpu.make_async_copy(er_hbm.at[dv], ev, esem).wait()

    fetch_idx(0, src_a, dst_a, isem_a)
    wait_idx(src_a, dst_a, isem_a)
    start_gathers(src_a, dst_a, hs_a, er_a, hsem_a, esem_a)
    fetch_idx(1, src_b, dst_b, isem_b)

    def pair_body(j, carry):
        wait_idx(src_b, dst_b, isem_b)
        start_gathers(src_b, dst_b, hs_b, er_b, hsem_b, esem_b)
        wait_gathers(src_a, dst_a, hs_a, er_a, hsem_a, esem_a)
        _edge_compute(hs_a, er_a, out_v)
        pltpu.sync_copy(out_v, acc.at[dst_a], add=True)
        fetch_idx(2 * j + 2, src_a, dst_a, isem_a)

        wait_idx(src_a, dst_a, isem_a)
        start_gathers(src_a, dst_a, hs_a, er_a, hsem_a, esem_a)
        wait_gathers(src_b, dst_b, hs_b, er_b, hsem_b, esem_b)
        _edge_compute(hs_b, er_b, out_v)
        pltpu.sync_copy(out_v, acc.at[dst_b], add=True)
        fetch_idx(2 * j + 3, src_b, dst_b, isem_b)
        return carry

    lax.fori_loop(0, NCHUNKS // 2, pair_body, 0)

    wait_idx(src_b, dst_b, isem_b)
    wait_gathers(src_a, dst_a, hs_a, er_a, hsem_a, esem_a)
    _edge_compute(hs_a, er_a, out_v)
    pltpu.sync_copy(out_v, acc.at[dst_a], add=True)

    plsc.subcore_barrier()
    pltpu.sync_copy(acc.at[pl.ds(sid * RPT, RPT)],
                    out_hbm.at[pl.ds(cid * N + sid * RPT, RPT)])


@jax.jit
def kernel(x, edge_index, W, attn_l, attn_r):
    srcp = edge_index[0].astype(jnp.int32)
    dstp = edge_index[1].astype(jnp.int32)

    bn = 1000
    grid = N // bn
    hs, er = pl.pallas_call(
        _prep_body,
        grid=(grid,),
        in_specs=[
            pl.BlockSpec((bn, F), lambda i: (i, 0)),
            pl.BlockSpec((F, HD), lambda i: (0, 0)),
            pl.BlockSpec((1, F), lambda i: (0, 0)),
            pl.BlockSpec((1, F), lambda i: (0, 0)),
        ],
        out_specs=[
            pl.BlockSpec((bn, ROW), lambda i: (i, 0)),
            pl.BlockSpec((bn, 2 * H), lambda i: (i, 0)),
        ],
        out_shape=[
            jax.ShapeDtypeStruct((N, ROW), jnp.float32),
            jax.ShapeDtypeStruct((N, 2 * H), jnp.float32),
        ],
    )(x, W, attn_l.reshape(1, F), attn_r.reshape(1, F))

    mesh = plsc.VectorSubcoreMesh(core_axis_name="c", subcore_axis_name="s",
                                  num_cores=NC, num_subcores=NS)
    partials = pl.kernel(
        _edge_body,
        out_type=jax.ShapeDtypeStruct((NC * N, ROW), jnp.float32),
        mesh=mesh,
        scratch_types=(
            [pltpu.VMEM((CHUNK,), jnp.int32),
             pltpu.VMEM((CHUNK,), jnp.int32),
             pltpu.VMEM((CHUNK, ROW), jnp.float32),
             pltpu.VMEM((CHUNK, 2 * H), jnp.float32)] * 2 +
            [pltpu.VMEM((CHUNK, ROW), jnp.float32),
             pltpu.VMEM_SHARED((N, ROW), jnp.float32),
             pltpu.SemaphoreType.DMA,
             pltpu.SemaphoreType.DMA,
             pltpu.SemaphoreType.DMA,
             pltpu.SemaphoreType.DMA,
             pltpu.SemaphoreType.DMA,
             pltpu.SemaphoreType.DMA]
        ),
        compiler_params=pltpu.CompilerParams(use_tc_tiling_on_sc=False),
    )(srcp, dstp, hs, er)

    feat = pl.pallas_call(
        _fin_body,
        grid=(grid,),
        in_specs=[
            pl.BlockSpec((bn, ROW), lambda i: (i, 0)),
            pl.BlockSpec((bn, ROW), lambda i: (i + grid, 0)),
        ],
        out_specs=pl.BlockSpec((bn, HD), lambda i: (i, 0)),
        out_shape=jax.ShapeDtypeStruct((N, HD), jnp.float32),
    )(partials, partials)
    return feat
